# Initial kernel scaffold; baseline (speedup 1.0000x reference)
import jax
import jax.numpy as jnp
from jax import lax
from jax.experimental import pallas as pl
from jax.experimental.pallas import tpu as pltpu
from jax.experimental.pallas import tpu_sc as plsc

HID = 256
HALF = 128
CHUNK = 128
NSUB = 16
NCORE = 2
N_AUTHOR, N_INST, N_DOMAIN, N_PAPER = 10000, 1000, 500, 10000
NP_AUTHOR, NP_INST, NP_DOMAIN, NP_PAPER = 10240, 1024, 512, 10240
F32 = jnp.float32


def _seg_sum(src, dst, x0, x1, n_dst_pad, want_count):
    e_pad = src.shape[0]
    per_sub = e_pad // NSUB
    n_chunks = per_sub // CHUNK
    rows_per_sub = n_dst_pad // NSUB

    outs = [jax.ShapeDtypeStruct((2 * n_dst_pad, HALF), F32)]
    if want_count:
        outs.append(jax.ShapeDtypeStruct((n_dst_pad, 16), F32))
    scratch = [
        pltpu.VMEM((CHUNK,), jnp.int32),
        pltpu.VMEM((CHUNK,), jnp.int32),
        pltpu.VMEM((CHUNK, HALF), F32),
        pltpu.VMEM((CHUNK, 16), F32),
        pltpu.VMEM_SHARED((n_dst_pad, HALF), F32),
        pltpu.VMEM_SHARED((n_dst_pad, 16), F32),
        pltpu.SemaphoreType.DMA,
    ]

    def body(src_ref, dst_ref, x0_ref, x1_ref, *rest):
        if want_count:
            out_ref, cnt_ref, sidx, didx, rows, ones, acc, cacc, sem = rest
        else:
            out_ref, sidx, didx, rows, ones, acc, cacc, sem = rest
        c = lax.axis_index("c")
        s = lax.axis_index("s")

        zero16 = jnp.zeros((16,), F32)
        one16 = jnp.full((16,), 1.0, F32)

        @pl.loop(0, CHUNK)
        def _(r):
            for k in range(HALF // 16):
                rows[r, pl.ds(k * 16, 16)] = zero16
            ones[r, pl.ds(0, 16)] = zero16

        r0 = s * rows_per_sub
        off = 0
        while off < rows_per_sub:
            m = min(CHUNK, rows_per_sub - off)
            pltpu.sync_copy(rows.at[pl.ds(0, m)], acc.at[pl.ds(r0 + off, m)])
            if want_count:
                @pl.when(c == 0)
                def _():
                    pltpu.sync_copy(ones.at[pl.ds(0, m)],
                                    cacc.at[pl.ds(r0 + off, m)])
            off += m

        if want_count:
            @pl.loop(0, CHUNK)
            def _(r):
                ones[r, pl.ds(0, 16)] = one16

        plsc.subcore_barrier()

        base0 = s * per_sub

        @pl.loop(0, n_chunks)
        def _(i):
            bb = base0 + i * CHUNK
            pltpu.sync_copy(src_ref.at[pl.ds(bb, CHUNK)], sidx)
            pltpu.sync_copy(dst_ref.at[pl.ds(bb, CHUNK)], didx)

            @pl.when(c == 0)
            def _():
                pltpu.async_copy(x0_ref.at[sidx], rows, sem).wait()

            @pl.when(c == 1)
            def _():
                pltpu.async_copy(x1_ref.at[sidx], rows, sem).wait()

            pltpu.sync_copy(rows, acc.at[didx], add=True)
            if want_count:
                @pl.when(c == 0)
                def _():
                    pltpu.sync_copy(ones, cacc.at[didx], add=True)

        plsc.subcore_barrier()
        pltpu.sync_copy(acc.at[pl.ds(r0, rows_per_sub)],
                        out_ref.at[pl.ds(c * n_dst_pad + r0, rows_per_sub)])
        if want_count:
            @pl.when(c == 0)
            def _():
                pltpu.sync_copy(cacc.at[pl.ds(r0, rows_per_sub)],
                                cnt_ref.at[pl.ds(r0, rows_per_sub)])

    fn = pl.kernel(
        body,
        out_type=outs,
        mesh=plsc.VectorSubcoreMesh(core_axis_name="c", subcore_axis_name="s"),
        scratch_types=scratch,
    )
    res = fn(src, dst, x0, x1)
    if want_count:
        sums, cnt = res
        return sums.reshape(2, n_dst_pad, HALF), cnt[:, :1]
    return res[0].reshape(2, n_dst_pad, HALF)


def _gather_pairs(h2a, h2p, l0, l1):
    e_pad = l0.shape[0]
    per_w = e_pad // (NCORE * NSUB)
    n_chunks = per_w // CHUNK

    def body(a_ref, p_ref, l0_ref, l1_ref, ag_ref, pg_ref, idx, rows, sem):
        c = lax.axis_index("c")
        s = lax.axis_index("s")
        w = s * NCORE + c

        @pl.loop(0, n_chunks)
        def _(i):
            b = w * per_w + i * CHUNK
            pltpu.sync_copy(l0_ref.at[pl.ds(b, CHUNK)], idx)
            pltpu.async_copy(a_ref.at[idx], rows, sem).wait()
            pltpu.sync_copy(rows, ag_ref.at[pl.ds(b, CHUNK)])
            pltpu.sync_copy(l1_ref.at[pl.ds(b, CHUNK)], idx)
            pltpu.async_copy(p_ref.at[idx], rows, sem).wait()
            pltpu.sync_copy(rows, pg_ref.at[pl.ds(b, CHUNK)])

    fn = pl.kernel(
        body,
        out_type=[jax.ShapeDtypeStruct((e_pad, HID), F32),
                  jax.ShapeDtypeStruct((e_pad, HID), F32)],
        mesh=plsc.VectorSubcoreMesh(core_axis_name="c", subcore_axis_name="s"),
        scratch_types=[
            pltpu.VMEM((CHUNK,), jnp.int32),
            pltpu.VMEM((CHUNK, HID), F32),
            pltpu.SemaphoreType.DMA,
        ],
    )
    return fn(h2a, h2p, l0, l1)


def _dense_layer(x0, x1, terms, w_root, bias, relu, out_full):
    n_pad = x0.shape[0]
    blk = 1280 if n_pad >= 1280 else n_pad
    grid = n_pad // blk

    args = [x0, x1]
    in_specs = [pl.BlockSpec((blk, HALF), lambda i: (i, 0)),
                pl.BlockSpec((blk, HALF), lambda i: (i, 0))]
    for sums, cnt, wn in terms:
        args += [sums, cnt, wn]
        in_specs += [
            pl.BlockSpec((2, blk, HALF), lambda i: (0, i, 0)),
            pl.BlockSpec((blk, 1), lambda i: (i, 0)),
            pl.BlockSpec((HID, HID), lambda i: (0, 0)),
        ]
    args += [w_root, bias]
    in_specs += [pl.BlockSpec((HID, HID), lambda i: (0, 0)),
                 pl.BlockSpec((1, HID), lambda i: (0, 0))]

    n_terms = len(terms)

    def body(*refs):
        x0_ref, x1_ref = refs[0], refs[1]
        wr_ref, b_ref = refs[2 + 3 * n_terms], refs[3 + 3 * n_terms]
        out_refs = refs[4 + 3 * n_terms:]
        x = jnp.concatenate([x0_ref[...], x1_ref[...]], axis=1)
        acc = jnp.dot(x, wr_ref[...], preferred_element_type=F32)
        for t in range(n_terms):
            s_ref, c_ref, wn_ref = refs[2 + 3 * t], refs[3 + 3 * t], refs[4 + 3 * t]
            inv = 1.0 / jnp.maximum(c_ref[...], 1.0)
            m = jnp.concatenate([s_ref[0], s_ref[1]], axis=1) * inv
            acc = acc + jnp.dot(m, wn_ref[...], preferred_element_type=F32)
        acc = acc + b_ref[...]
        if relu:
            acc = jnp.maximum(acc, 0.0)
        if out_full:
            out_refs[0][...] = acc
        else:
            out_refs[0][...] = acc[:, :HALF]
            out_refs[1][...] = acc[:, HALF:]

    if out_full:
        out_shape = [jax.ShapeDtypeStruct((n_pad, HID), F32)]
        out_specs = [pl.BlockSpec((blk, HID), lambda i: (i, 0))]
    else:
        out_shape = [jax.ShapeDtypeStruct((n_pad, HALF), F32),
                     jax.ShapeDtypeStruct((n_pad, HALF), F32)]
        out_specs = [pl.BlockSpec((blk, HALF), lambda i: (i, 0)),
                     pl.BlockSpec((blk, HALF), lambda i: (i, 0))]

    res = pl.pallas_call(
        body,
        grid=(grid,),
        in_specs=in_specs,
        out_specs=out_specs,
        out_shape=out_shape,
    )(*args)
    return res[0] if out_full else res


def _row_dot(a, p):
    e_pad = a.shape[0]
    blk = 2048
    grid = e_pad // blk

    def body(a_ref, p_ref, o_ref):
        o_ref[...] = jnp.sum(a_ref[...] * p_ref[...], axis=1, keepdims=True)

    return pl.pallas_call(
        body,
        grid=(grid,),
        in_specs=[pl.BlockSpec((blk, HID), lambda i: (i, 0)),
                  pl.BlockSpec((blk, HID), lambda i: (i, 0))],
        out_specs=pl.BlockSpec((blk, 1), lambda i: (i, 0)),
        out_shape=jax.ShapeDtypeStruct((e_pad, 1), F32),
    )(a, p)


def _pad_rows(x, n_pad):
    xp = jnp.pad(x, ((0, n_pad - x.shape[0]), (0, 0)))
    return xp[:, :HALF], xp[:, HALF:]


def _pad_edges(e_src, e_dst, e_pad, trash):
    n = e_src.shape[0]
    src = jnp.concatenate(
        [e_src.astype(jnp.int32), jnp.zeros((e_pad - n,), jnp.int32)])
    dst = jnp.concatenate(
        [e_dst.astype(jnp.int32), jnp.full((e_pad - n,), trash, jnp.int32)])
    return src, dst


def kernel(author_n_id, institution_n_id, domain_n_id, paper_n_id,
           edge_index_writes, edge_index_affiliated, edge_index_has_topic,
           edge_label_index,
           emb_author, emb_institution, emb_domain, emb_paper,
           W_root, W_neigh, b):
    a0, a1 = _pad_rows(emb_author, NP_AUTHOR)
    p0, p1 = _pad_rows(emb_paper, NP_PAPER)
    i0, i1 = _pad_rows(emb_institution, NP_INST)
    d0, d1 = _pad_rows(emb_domain, NP_DOMAIN)

    EP_W, EP_A, EP_T, EP_L = 161792, 20480, 40960, 20480
    ws, wd = edge_index_writes[0], edge_index_writes[1]
    asrc, adst = edge_index_affiliated[0], edge_index_affiliated[1]
    ts, td = edge_index_has_topic[0], edge_index_has_topic[1]

    e_ap = _pad_edges(ws, wd, EP_W, N_PAPER)
    e_pa = _pad_edges(wd, ws, EP_W, N_AUTHOR)
    e_ai = _pad_edges(asrc, adst, EP_A, N_INST)
    e_ia = _pad_edges(adst, asrc, EP_A, N_AUTHOR)
    e_pd = _pad_edges(ts, td, EP_T, N_DOMAIN)
    e_dp = _pad_edges(td, ts, EP_T, N_PAPER)

    s_ap, c_ap = _seg_sum(*e_ap, a0, a1, NP_PAPER, True)
    s_ai, c_ai = _seg_sum(*e_ai, a0, a1, NP_INST, True)
    s_pd, c_pd = _seg_sum(*e_pd, p0, p1, NP_DOMAIN, True)
    s_pa, c_pa = _seg_sum(*e_pa, p0, p1, NP_AUTHOR, True)
    s_ia, c_ia = _seg_sum(*e_ia, i0, i1, NP_AUTHOR, True)
    s_dp, c_dp = _seg_sum(*e_dp, d0, d1, NP_PAPER, True)

    wr0, wn0, b0 = W_root[0], W_neigh[0], b[0]
    wr1, wn1, b1 = W_root[1], W_neigh[1], b[1]

    h1p0, h1p1 = _dense_layer(
        p0, p1, [(s_ap, c_ap, wn0[0]), (s_dp, c_dp, wn0[5])],
        wr0[0] + wr0[5], (b0[0] + b0[5])[None, :], True, False)
    h1a0, h1a1 = _dense_layer(
        a0, a1, [(s_pa, c_pa, wn0[3]), (s_ia, c_ia, wn0[4])],
        wr0[3] + wr0[4], (b0[3] + b0[4])[None, :], True, False)
    h1i0, h1i1 = _dense_layer(
        i0, i1, [(s_ai, c_ai, wn0[1])], wr0[1], b0[1][None, :], True, False)
    h1d0, h1d1 = _dense_layer(
        d0, d1, [(s_pd, c_pd, wn0[2])], wr0[2], b0[2][None, :], True, False)

    s2_ap = _seg_sum(*e_ap, h1a0, h1a1, NP_PAPER, False)
    s2_dp = _seg_sum(*e_dp, h1d0, h1d1, NP_PAPER, False)
    s2_pa = _seg_sum(*e_pa, h1p0, h1p1, NP_AUTHOR, False)
    s2_ia = _seg_sum(*e_ia, h1i0, h1i1, NP_AUTHOR, False)

    h2p = _dense_layer(
        h1p0, h1p1, [(s2_ap, c_ap, wn1[0]), (s2_dp, c_dp, wn1[5])],
        wr1[0] + wr1[5], (b1[0] + b1[5])[None, :], False, True)
    h2a = _dense_layer(
        h1a0, h1a1, [(s2_pa, c_pa, wn1[3]), (s2_ia, c_ia, wn1[4])],
        wr1[3] + wr1[4], (b1[3] + b1[4])[None, :], False, True)

    l0 = jnp.concatenate([edge_label_index[0].astype(jnp.int32),
                          jnp.zeros((EP_L - 20000,), jnp.int32)])
    l1 = jnp.concatenate([edge_label_index[1].astype(jnp.int32),
                          jnp.zeros((EP_L - 20000,), jnp.int32)])
    ag, pg = _gather_pairs(h2a, h2p, l0, l1)
    pred = _row_dot(ag, pg)
    return pred[:20000, 0]


# speedup vs baseline: 2.0204x; 2.0204x over previous
import jax
import jax.numpy as jnp
from jax import lax
from jax.experimental import pallas as pl
from jax.experimental.pallas import tpu as pltpu
from jax.experimental.pallas import tpu_sc as plsc

HID = 256
HALF = 128
CHUNK = 128
NSUB = 16
NCORE = 2
N_AUTHOR, N_INST, N_DOMAIN, N_PAPER = 10000, 1000, 500, 10000
NP_AUTHOR, NP_INST, NP_DOMAIN, NP_PAPER = 10240, 1024, 512, 10240
F32 = jnp.float32


def _seg_sum(src, dst, x_both, n_dst_pad):
    e_pad = src.shape[0]
    n_src_pad = x_both.shape[0] // 2
    per_sub = e_pad // NSUB
    n_chunks = per_sub // CHUNK
    rows_per_sub = n_dst_pad // NSUB

    def body(src_ref, dst_ref, x_ref, out_ref, sidx, didx, rows, acc, sem):
        c = lax.axis_index("c")
        s = lax.axis_index("s")
        src_off = c * n_src_pad

        zero16 = jnp.zeros((16,), F32)

        @pl.loop(0, CHUNK)
        def _(r):
            for k in range(HALF // 16):
                rows[r, pl.ds(k * 16, 16)] = zero16

        r0 = s * rows_per_sub
        off = 0
        while off < rows_per_sub:
            m = min(CHUNK, rows_per_sub - off)
            pltpu.sync_copy(rows.at[pl.ds(0, m)], acc.at[pl.ds(r0 + off, m)])
            off += m

        plsc.subcore_barrier()

        base0 = s * per_sub

        @pl.loop(0, n_chunks)
        def _(i):
            bb = base0 + i * CHUNK
            pltpu.sync_copy(src_ref.at[pl.ds(bb, CHUNK)], sidx)
            pltpu.sync_copy(dst_ref.at[pl.ds(bb, CHUNK)], didx)
            for k in range(CHUNK // 16):
                sl = pl.ds(k * 16, 16)
                sidx[sl] = sidx[sl] + src_off
            pltpu.async_copy(x_ref.at[sidx], rows, sem).wait()
            pltpu.sync_copy(rows, acc.at[didx], add=True)

        plsc.subcore_barrier()
        pltpu.sync_copy(acc.at[pl.ds(r0, rows_per_sub)],
                        out_ref.at[pl.ds(c * n_dst_pad + r0, rows_per_sub)])

    fn = pl.kernel(
        body,
        out_type=[jax.ShapeDtypeStruct((2 * n_dst_pad, HALF), F32)],
        mesh=plsc.VectorSubcoreMesh(core_axis_name="c", subcore_axis_name="s"),
        scratch_types=[
            pltpu.VMEM((CHUNK,), jnp.int32),
            pltpu.VMEM((CHUNK,), jnp.int32),
            pltpu.VMEM((CHUNK, HALF), F32),
            pltpu.VMEM_SHARED((n_dst_pad, HALF), F32),
            pltpu.SemaphoreType.DMA,
        ],
    )
    return fn(src, dst, x_both)[0].reshape(2, n_dst_pad, HALF)


HSIZE = 16384


def _degree_counts(dst, n_dst_pad):
    e_pad = dst.shape[0]
    per_sub = e_pad // NSUB
    n_chunks = per_sub // CHUNK
    cols = HSIZE // NSUB

    def body(dst_ref, out_ref, didx, hist, buf, accv, sgrid):
        s = lax.axis_index("s")

        zero16 = jnp.zeros((16,), F32)
        one16 = jnp.full((16,), 1.0, F32)

        @pl.loop(0, HSIZE // 16)
        def _(i):
            hist[pl.ds(i * 16, 16)] = zero16

        @pl.loop(0, n_chunks)
        def _(i):
            bb = s * per_sub + i * CHUNK
            pltpu.sync_copy(dst_ref.at[pl.ds(bb, CHUNK)], didx)
            for k in range(CHUNK // 16):
                plsc.addupdate_scatter(hist, [didx[pl.ds(k * 16, 16)]], one16)

        pltpu.sync_copy(hist, sgrid.at[s])
        plsc.subcore_barrier()
        col0 = s * cols
        for t in range(NSUB):
            pltpu.sync_copy(sgrid.at[t, pl.ds(col0, cols)],
                            buf.at[pl.ds(t * cols, cols)])

        @pl.loop(0, cols // 16)
        def _(k):
            sl = pl.ds(k * 16, 16)
            v = buf[sl]
            for t in range(1, NSUB):
                v = v + buf[pl.ds(t * cols + k * 16, 16)]
            accv[sl] = v

        pltpu.sync_copy(accv, out_ref.at[pl.ds(s * cols, cols)])

    fn = pl.kernel(
        body,
        out_type=[jax.ShapeDtypeStruct((HSIZE,), F32)],
        mesh=plsc.VectorSubcoreMesh(core_axis_name="c", subcore_axis_name="s"),
        scratch_types=[
            pltpu.VMEM((CHUNK,), jnp.int32),
            pltpu.VMEM((HSIZE,), F32),
            pltpu.VMEM((NSUB * (HSIZE // NSUB),), F32),
            pltpu.VMEM((HSIZE // NSUB,), F32),
            pltpu.VMEM_SHARED((NSUB, HSIZE), F32),
        ],
        compiler_params=pltpu.CompilerParams(needs_layout_passes=False),
    )
    return fn(dst)[0][:n_dst_pad, None]


def _gather_pairs(h2a, h2p, l0, l1):
    e_pad = l0.shape[0]
    per_w = e_pad // (NCORE * NSUB)
    n_chunks = per_w // CHUNK

    def body(a_ref, p_ref, l0_ref, l1_ref, ag_ref, pg_ref, idx, rows, sem):
        c = lax.axis_index("c")
        s = lax.axis_index("s")
        w = s * NCORE + c

        @pl.loop(0, n_chunks)
        def _(i):
            b = w * per_w + i * CHUNK
            pltpu.sync_copy(l0_ref.at[pl.ds(b, CHUNK)], idx)
            pltpu.async_copy(a_ref.at[idx], rows, sem).wait()
            pltpu.sync_copy(rows, ag_ref.at[pl.ds(b, CHUNK)])
            pltpu.sync_copy(l1_ref.at[pl.ds(b, CHUNK)], idx)
            pltpu.async_copy(p_ref.at[idx], rows, sem).wait()
            pltpu.sync_copy(rows, pg_ref.at[pl.ds(b, CHUNK)])

    fn = pl.kernel(
        body,
        out_type=[jax.ShapeDtypeStruct((e_pad, HID), F32),
                  jax.ShapeDtypeStruct((e_pad, HID), F32)],
        mesh=plsc.VectorSubcoreMesh(core_axis_name="c", subcore_axis_name="s"),
        scratch_types=[
            pltpu.VMEM((CHUNK,), jnp.int32),
            pltpu.VMEM((CHUNK, HID), F32),
            pltpu.SemaphoreType.DMA,
        ],
    )
    return fn(h2a, h2p, l0, l1)


def _dense_layer(x0, x1, terms, w_root, bias, relu, out_full):
    n_pad = x0.shape[0]
    blk = 1280 if n_pad >= 1280 else n_pad
    grid = n_pad // blk

    args = [x0, x1]
    in_specs = [pl.BlockSpec((blk, HALF), lambda i: (i, 0)),
                pl.BlockSpec((blk, HALF), lambda i: (i, 0))]
    for sums, cnt, wn in terms:
        args += [sums, cnt, wn]
        in_specs += [
            pl.BlockSpec((2, blk, HALF), lambda i: (0, i, 0)),
            pl.BlockSpec((blk, 1), lambda i: (i, 0)),
            pl.BlockSpec((HID, HID), lambda i: (0, 0)),
        ]
    args += [w_root, bias]
    in_specs += [pl.BlockSpec((HID, HID), lambda i: (0, 0)),
                 pl.BlockSpec((1, HID), lambda i: (0, 0))]

    n_terms = len(terms)

    def body(*refs):
        x0_ref, x1_ref = refs[0], refs[1]
        wr_ref, b_ref = refs[2 + 3 * n_terms], refs[3 + 3 * n_terms]
        out_refs = refs[4 + 3 * n_terms:]
        x = jnp.concatenate([x0_ref[...], x1_ref[...]], axis=1)
        acc = jnp.dot(x, wr_ref[...], preferred_element_type=F32)
        for t in range(n_terms):
            s_ref, c_ref, wn_ref = refs[2 + 3 * t], refs[3 + 3 * t], refs[4 + 3 * t]
            inv = 1.0 / jnp.maximum(c_ref[...], 1.0)
            m = jnp.concatenate([s_ref[0], s_ref[1]], axis=1) * inv
            acc = acc + jnp.dot(m, wn_ref[...], preferred_element_type=F32)
        acc = acc + b_ref[...]
        if relu:
            acc = jnp.maximum(acc, 0.0)
        if out_full:
            out_refs[0][...] = acc
        else:
            out_refs[0][0] = acc[:, :HALF]
            out_refs[0][1] = acc[:, HALF:]

    if out_full:
        out_shape = [jax.ShapeDtypeStruct((n_pad, HID), F32)]
        out_specs = [pl.BlockSpec((blk, HID), lambda i: (i, 0))]
    else:
        out_shape = [jax.ShapeDtypeStruct((2, n_pad, HALF), F32)]
        out_specs = [pl.BlockSpec((2, blk, HALF), lambda i: (0, i, 0))]

    res = pl.pallas_call(
        body,
        grid=(grid,),
        in_specs=in_specs,
        out_specs=out_specs,
        out_shape=out_shape,
    )(*args)
    return res[0]


def _row_dot(a, p):
    e_pad = a.shape[0]
    blk = 2048
    grid = e_pad // blk

    def body(a_ref, p_ref, o_ref):
        o_ref[...] = jnp.sum(a_ref[...] * p_ref[...], axis=1, keepdims=True)

    return pl.pallas_call(
        body,
        grid=(grid,),
        in_specs=[pl.BlockSpec((blk, HID), lambda i: (i, 0)),
                  pl.BlockSpec((blk, HID), lambda i: (i, 0))],
        out_specs=pl.BlockSpec((blk, 1), lambda i: (i, 0)),
        out_shape=jax.ShapeDtypeStruct((e_pad, 1), F32),
    )(a, p)


def _pad_rows(x, n_pad):
    xp = jnp.pad(x, ((0, n_pad - x.shape[0]), (0, 0)))
    return xp[:, :HALF], xp[:, HALF:]


def _pad_edges(e_src, e_dst, e_pad, trash):
    n = e_src.shape[0]
    src = jnp.concatenate(
        [e_src.astype(jnp.int32), jnp.zeros((e_pad - n,), jnp.int32)])
    dst = jnp.concatenate(
        [e_dst.astype(jnp.int32), jnp.full((e_pad - n,), trash, jnp.int32)])
    return src, dst


def kernel(author_n_id, institution_n_id, domain_n_id, paper_n_id,
           edge_index_writes, edge_index_affiliated, edge_index_has_topic,
           edge_label_index,
           emb_author, emb_institution, emb_domain, emb_paper,
           W_root, W_neigh, b):
    a0, a1 = _pad_rows(emb_author, NP_AUTHOR)
    p0, p1 = _pad_rows(emb_paper, NP_PAPER)
    i0, i1 = _pad_rows(emb_institution, NP_INST)
    d0, d1 = _pad_rows(emb_domain, NP_DOMAIN)

    EP_W, EP_A, EP_T, EP_L = 161792, 20480, 40960, 20480
    ws, wd = edge_index_writes[0], edge_index_writes[1]
    asrc, adst = edge_index_affiliated[0], edge_index_affiliated[1]
    ts, td = edge_index_has_topic[0], edge_index_has_topic[1]

    e_ap = _pad_edges(ws, wd, EP_W, N_PAPER)
    e_pa = _pad_edges(wd, ws, EP_W, N_AUTHOR)
    e_ai = _pad_edges(asrc, adst, EP_A, N_INST)
    e_ia = _pad_edges(adst, asrc, EP_A, N_AUTHOR)
    e_pd = _pad_edges(ts, td, EP_T, N_DOMAIN)
    e_dp = _pad_edges(td, ts, EP_T, N_PAPER)

    a_both = jnp.concatenate([a0, a1], axis=0)
    p_both = jnp.concatenate([p0, p1], axis=0)
    i_both = jnp.concatenate([i0, i1], axis=0)
    d_both = jnp.concatenate([d0, d1], axis=0)

    c_ap = _degree_counts(e_ap[1], NP_PAPER)
    c_ai = _degree_counts(e_ai[1], NP_INST)
    c_pd = _degree_counts(e_pd[1], NP_DOMAIN)
    c_pa = _degree_counts(e_pa[1], NP_AUTHOR)
    c_ia = _degree_counts(e_ia[1], NP_AUTHOR)
    c_dp = _degree_counts(e_dp[1], NP_PAPER)

    s_ap = _seg_sum(*e_ap, a_both, NP_PAPER)
    s_ai = _seg_sum(*e_ai, a_both, NP_INST)
    s_pd = _seg_sum(*e_pd, p_both, NP_DOMAIN)
    s_pa = _seg_sum(*e_pa, p_both, NP_AUTHOR)
    s_ia = _seg_sum(*e_ia, i_both, NP_AUTHOR)
    s_dp = _seg_sum(*e_dp, d_both, NP_PAPER)

    wr0, wn0, b0 = W_root[0], W_neigh[0], b[0]
    wr1, wn1, b1 = W_root[1], W_neigh[1], b[1]

    h1p = _dense_layer(
        p0, p1, [(s_ap, c_ap, wn0[0]), (s_dp, c_dp, wn0[5])],
        wr0[0] + wr0[5], (b0[0] + b0[5])[None, :], True, False)
    h1a = _dense_layer(
        a0, a1, [(s_pa, c_pa, wn0[3]), (s_ia, c_ia, wn0[4])],
        wr0[3] + wr0[4], (b0[3] + b0[4])[None, :], True, False)
    h1i = _dense_layer(
        i0, i1, [(s_ai, c_ai, wn0[1])], wr0[1], b0[1][None, :], True, False)
    h1d = _dense_layer(
        d0, d1, [(s_pd, c_pd, wn0[2])], wr0[2], b0[2][None, :], True, False)

    s2_ap = _seg_sum(*e_ap, h1a.reshape(2 * NP_AUTHOR, HALF), NP_PAPER)
    s2_dp = _seg_sum(*e_dp, h1d.reshape(2 * NP_DOMAIN, HALF), NP_PAPER)
    s2_pa = _seg_sum(*e_pa, h1p.reshape(2 * NP_PAPER, HALF), NP_AUTHOR)
    s2_ia = _seg_sum(*e_ia, h1i.reshape(2 * NP_INST, HALF), NP_AUTHOR)

    h2p = _dense_layer(
        h1p[0], h1p[1], [(s2_ap, c_ap, wn1[0]), (s2_dp, c_dp, wn1[5])],
        wr1[0] + wr1[5], (b1[0] + b1[5])[None, :], False, True)
    h2a = _dense_layer(
        h1a[0], h1a[1], [(s2_pa, c_pa, wn1[3]), (s2_ia, c_ia, wn1[4])],
        wr1[3] + wr1[4], (b1[3] + b1[4])[None, :], False, True)

    l0 = jnp.concatenate([edge_label_index[0].astype(jnp.int32),
                          jnp.zeros((EP_L - 20000,), jnp.int32)])
    l1 = jnp.concatenate([edge_label_index[1].astype(jnp.int32),
                          jnp.zeros((EP_L - 20000,), jnp.int32)])
    ag, pg = _gather_pairs(h2a, h2p, l0, l1)
    pred = _row_dot(ag, pg)
    return pred[:20000, 0]
